# baseline (speedup 1.0000x reference)
import functools

import jax
import jax.numpy as jnp
from jax import lax
from jax.experimental import pallas as pl
from jax.experimental.pallas import tpu as pltpu
from jax.experimental.pallas import tpu_sc as plsc

D = 768
L = 16
NK = D // L
N_TOK = 4096 * 32
K_ROWS = 64


def _rsqrt_vec(x):
    i = plsc.bitcast(x, jnp.int32)
    i = jnp.int32(0x5F3759DF) - lax.shift_right_logical(i, 1)
    y = plsc.bitcast(i, jnp.float32)
    for _ in range(3):
        y = y * (1.5 - 0.5 * x * y * y)
    return y


def _sc_body(n_workers, x_hbm, seg_hbm, tok_hbm, pos_hbm, segtab_hbm,
             gam_hbm, bet_hbm, out_hbm,
             idxc, segc, buf, posp, dvec, gam, bet, stage, sem):
    tpw = N_TOK // n_workers
    n_chunks = tpw // K_ROWS
    wid = lax.axis_index("s") * 2 + lax.axis_index("c")
    base = wid * tpw

    pltpu.sync_copy(gam_hbm, gam)
    pltpu.sync_copy(bet_hbm, bet)
    pltpu.sync_copy(segtab_hbm, stage)
    pltpu.sync_copy(pos_hbm, posp)

    for k in range(NK):
        sl = pl.ds(k * L, L)
        dvec[sl] = stage[pl.ds(D + k * L, L)] - stage[sl]

    def posfix(t, carry):
        for k in range(NK):
            sl = pl.ds(t * D + k * L, L)
            posp[sl] = posp[sl] + stage[pl.ds(k * L, L)]
        return carry

    lax.fori_loop(0, 32, posfix, 0)

    def chunk_body(c, carry):
        rowbase = base + c * K_ROWS
        pltpu.sync_copy(x_hbm.at[pl.ds(rowbase, K_ROWS)], idxc)
        pltpu.sync_copy(seg_hbm.at[pl.ds(rowbase, K_ROWS)], segc)
        copies = []
        for j in range(K_ROWS // L):
            iv = idxc[pl.ds(j * L, L)]
            copies.append(
                pltpu.async_copy(tok_hbm.at[iv], buf.at[pl.ds(j * L, L)], sem))
        for cp in copies:
            cp.wait()

        @plsc.parallel_loop(0, K_ROWS // 2, unroll=2)
        def row_body(r):
            rows = (r, r + 32)
            svs = [plsc.load_gather(segc, [jnp.full((L,), rw, jnp.int32)])
                   for rw in rows]
            accs = [jnp.zeros((L,), jnp.float32) for _ in rows]
            acc2s = [jnp.zeros((L,), jnp.float32) for _ in rows]
            for k in range(NK):
                sl = pl.ds(k * L, L)
                pv = posp[pl.ds(r * D + k * L, L)]
                dv = dvec[sl]
                for g in range(2):
                    tv = buf[rows[g], sl] + (pv + svs[g] * dv)
                    buf[rows[g], sl] = tv
                    accs[g] = accs[g] + tv
                    acc2s[g] = acc2s[g] + tv * tv
            means, invs = [], []
            for g in range(2):
                s1 = jnp.sum(accs[g])
                s2 = jnp.sum(acc2s[g])
                mean = jnp.full((L,), s1, jnp.float32) * (1.0 / D)
                ex2 = jnp.full((L,), s2, jnp.float32) * (1.0 / D)
                means.append(mean)
                invs.append(_rsqrt_vec(ex2 - mean * mean + 1e-5))
            for k in range(NK):
                sl = pl.ds(k * L, L)
                g_ = gam[sl]
                b_ = bet[sl]
                for g in range(2):
                    u = (buf[rows[g], sl] - means[g]) * invs[g]
                    buf[rows[g], sl] = u * g_ + b_

        pltpu.sync_copy(buf, out_hbm.at[pl.ds(rowbase, K_ROWS)])
        return carry

    lax.fori_loop(0, n_chunks, chunk_body, 0)


@jax.jit
def kernel(x, seg, tok_table, pos_table, seg_table, gamma, beta):
    info = plsc.get_sparse_core_info()
    n_workers = info.num_cores * info.num_subcores
    mesh = plsc.VectorSubcoreMesh(core_axis_name="c", subcore_axis_name="s")
    run = pl.kernel(
        functools.partial(_sc_body, n_workers),
        mesh=mesh,
        compiler_params=pltpu.CompilerParams(needs_layout_passes=False),
        out_type=jax.ShapeDtypeStruct((N_TOK, D), jnp.float32),
        scratch_types=[
            pltpu.VMEM((K_ROWS,), jnp.int32),
            pltpu.VMEM((K_ROWS,), jnp.float32),
            pltpu.VMEM((K_ROWS, D), jnp.float32),
            pltpu.VMEM((32 * D,), jnp.float32),
            pltpu.VMEM((D,), jnp.float32),
            pltpu.VMEM((D,), jnp.float32),
            pltpu.VMEM((D,), jnp.float32),
            pltpu.VMEM((2 * D,), jnp.float32),
            pltpu.SemaphoreType.DMA,
        ],
    )
    out = run(x.reshape(-1), seg.astype(jnp.float32).reshape(-1),
              tok_table, pos_table.reshape(-1), seg_table.reshape(-1),
              gamma, beta)
    return out.reshape(x.shape[0], x.shape[1], D)


# speedup vs baseline: 2.8510x; 2.8510x over previous
import functools

import jax
import jax.numpy as jnp
from jax import lax
from jax.experimental import pallas as pl
from jax.experimental.pallas import tpu as pltpu
from jax.experimental.pallas import tpu_sc as plsc

D = 768
L = 16
NK = D // L
N_TOK = 4096 * 32
K_ROWS = 32
NPART = 4
NCOMBO = 64


def _rsqrt_vec(x):
    i = plsc.bitcast(x, jnp.int32)
    i = jnp.int32(0x5F3759DF) - lax.shift_right_logical(i, 1)
    y = plsc.bitcast(i, jnp.float32)
    for _ in range(3):
        y = y * (1.5 - 0.5 * x * y * y)
    return y


def _sc_body(n_workers, x_hbm, seg_hbm, tok_hbm, pos_hbm, segtab_hbm,
             gam_hbm, bet_hbm, out_hbm, chbm,
             idxc, segc, cidxb, buf, cbuf, tbuf, ctmp, gam, bet, rowtmp,
             segrows, sem):
    tpw = N_TOK // n_workers
    n_chunks = tpw // K_ROWS
    wid = lax.axis_index("s") * 2 + lax.axis_index("c")
    base = wid * tpw
    cbase = wid * NCOMBO

    pltpu.sync_copy(gam_hbm, gam)
    pltpu.sync_copy(bet_hbm, bet)
    pltpu.sync_copy(segtab_hbm, segrows)

    def buildc(t, carry):
        pltpu.sync_copy(pos_hbm.at[pl.ds(t * D, D)], rowtmp)
        for s in range(2):
            row = s * 32 + t
            for k in range(NK):
                sl = pl.ds(k * L, L)
                ctmp[row, sl] = rowtmp[sl] + segrows[pl.ds(s * D + k * L, L)]
        return carry

    lax.fori_loop(0, 32, buildc, 0)
    pltpu.sync_copy(ctmp, chbm.at[pl.ds(cbase, NCOMBO)])

    lane_iota = lax.iota(jnp.int32, L)

    def chunk_body(c, carry):
        rowbase = base + c * K_ROWS
        pltpu.sync_copy(x_hbm.at[pl.ds(rowbase, K_ROWS)], idxc)
        pltpu.sync_copy(seg_hbm.at[pl.ds(rowbase, K_ROWS)], segc)
        for j in range(K_ROWS // L):
            cidxb[pl.ds(j * L, L)] = (
                segc[pl.ds(j * L, L)] * 32 + ((j & 1) * L + lane_iota)
                + cbase)
        cp1 = pltpu.async_copy(tok_hbm.at[idxc], buf, sem)
        cp2 = pltpu.async_copy(chbm.at[cidxb], cbuf, sem)
        cp1.wait()
        cp2.wait()

        def row_body(r, rcarry):
            rows = (r, r + 16)
            accs = [[jnp.zeros((L,), jnp.float32) for _ in range(NPART)]
                    for _ in rows]
            acc2s = [[jnp.zeros((L,), jnp.float32) for _ in range(NPART)]
                     for _ in rows]
            for k in range(NK):
                sl = pl.ds(k * L, L)
                p = k % NPART
                for g in range(2):
                    v = buf[rows[g], sl] + cbuf[rows[g], sl]
                    accs[g][p] = accs[g][p] + v
                    acc2s[g][p] = acc2s[g][p] + v * v
            means, invs = [], []
            for g in range(2):
                a = (accs[g][0] + accs[g][1]) + (accs[g][2] + accs[g][3])
                a2 = (acc2s[g][0] + acc2s[g][1]) + (acc2s[g][2] + acc2s[g][3])
                s1 = jnp.sum(a)
                s2 = jnp.sum(a2)
                mean = jnp.full((L,), s1, jnp.float32) * (1.0 / D)
                ex2 = jnp.full((L,), s2, jnp.float32) * (1.0 / D)
                means.append(mean)
                invs.append(_rsqrt_vec(ex2 - mean * mean + 1e-5))
            for k in range(NK):
                sl = pl.ds(k * L, L)
                g_ = gam[sl]
                b_ = bet[sl]
                for g in range(2):
                    v = buf[rows[g], sl] + cbuf[rows[g], sl]
                    u = (v - means[g]) * invs[g]
                    tbuf[rows[g], sl] = u * g_ + b_
            return rcarry

        lax.fori_loop(0, K_ROWS // 2, row_body, 0)
        pltpu.sync_copy(tbuf, out_hbm.at[pl.ds(rowbase, K_ROWS)])
        return carry

    lax.fori_loop(0, n_chunks, chunk_body, 0)


@jax.jit
def kernel(x, seg, tok_table, pos_table, seg_table, gamma, beta):
    info = plsc.get_sparse_core_info()
    n_workers = info.num_cores * info.num_subcores
    mesh = plsc.VectorSubcoreMesh(core_axis_name="c", subcore_axis_name="s")
    run = pl.kernel(
        functools.partial(_sc_body, n_workers),
        mesh=mesh,
        compiler_params=pltpu.CompilerParams(needs_layout_passes=False),
        out_type=(
            jax.ShapeDtypeStruct((N_TOK, D), jnp.float32),
            jax.ShapeDtypeStruct((n_workers * NCOMBO, D), jnp.float32),
        ),
        scratch_types=[
            pltpu.VMEM((K_ROWS,), jnp.int32),
            pltpu.VMEM((K_ROWS,), jnp.int32),
            pltpu.VMEM((K_ROWS,), jnp.int32),
            pltpu.VMEM((K_ROWS, D), jnp.float32),
            pltpu.VMEM((K_ROWS, D), jnp.float32),
            pltpu.VMEM((K_ROWS, D), jnp.float32),
            pltpu.VMEM((NCOMBO, D), jnp.float32),
            pltpu.VMEM((D,), jnp.float32),
            pltpu.VMEM((D,), jnp.float32),
            pltpu.VMEM((D,), jnp.float32),
            pltpu.VMEM((2 * D,), jnp.float32),
            pltpu.SemaphoreType.DMA,
        ],
    )
    out, _ = run(x.reshape(-1), seg.reshape(-1),
                 tok_table, pos_table.reshape(-1), seg_table.reshape(-1),
                 gamma, beta)
    return out.reshape(x.shape[0], x.shape[1], D)


# speedup vs baseline: 3.7254x; 1.3067x over previous
import functools

import jax
import jax.numpy as jnp
from jax import lax
from jax.experimental import pallas as pl
from jax.experimental.pallas import tpu as pltpu
from jax.experimental.pallas import tpu_sc as plsc

D = 768
L = 16
NK = D // L
N_TOK = 4096 * 32
K_ROWS = 32
NPART = 4
NCOMBO = 64


def _rsqrt_vec(x):
    i = plsc.bitcast(x, jnp.int32)
    i = jnp.int32(0x5F3759DF) - lax.shift_right_logical(i, 1)
    y = plsc.bitcast(i, jnp.float32)
    for _ in range(3):
        y = y * (1.5 - 0.5 * x * y * y)
    return y


def _sc_body(n_workers, x_hbm, seg_hbm, tok_hbm, pos_hbm, segtab_hbm,
             gam_hbm, bet_hbm, out_hbm, chbm,
             idxcA, segcA, cidxbA, bufA, cbufA,
             idxcB, segcB, cidxbB, bufB, cbufB,
             tbuf, gam, bet, rowtmp, segrows, semA, semB, semI):
    tpw = N_TOK // n_workers
    n_chunks = tpw // K_ROWS
    wid = lax.axis_index("s") * 2 + lax.axis_index("c")
    base = wid * tpw
    cbase = wid * NCOMBO
    last_rb = base + (n_chunks - 1) * K_ROWS
    lane_iota = lax.iota(jnp.int32, L)

    pltpu.sync_copy(gam_hbm, gam)
    pltpu.sync_copy(bet_hbm, bet)
    pltpu.sync_copy(segtab_hbm, segrows)

    def buildc(t, carry):
        pltpu.sync_copy(pos_hbm.at[pl.ds(t * D, D)], rowtmp)
        for s, dstref in ((0, bufA), (1, cbufA)):
            for k in range(NK):
                sl = pl.ds(k * L, L)
                dstref[t, sl] = rowtmp[sl] + segrows[pl.ds(s * D + k * L, L)]
        return carry

    lax.fori_loop(0, 32, buildc, 0)
    pltpu.sync_copy(bufA, chbm.at[pl.ds(cbase, K_ROWS)])
    pltpu.sync_copy(cbufA, chbm.at[pl.ds(cbase + K_ROWS, K_ROWS)])

    def stage_idx(rowbase, idxc, segc):
        c1 = pltpu.async_copy(x_hbm.at[pl.ds(rowbase, K_ROWS)], idxc, semI)
        c2 = pltpu.async_copy(seg_hbm.at[pl.ds(rowbase, K_ROWS)], segc, semI)
        return c1, c2

    def wait_idx(idxc, segc):
        pltpu.make_async_copy(x_hbm.at[pl.ds(0, K_ROWS)], idxc, semI).wait()
        pltpu.make_async_copy(seg_hbm.at[pl.ds(0, K_ROWS)], segc, semI).wait()

    def fire_gathers(idxc, segc, cidxb, buf, cbuf, sem):
        for j in range(K_ROWS // L):
            cidxb[pl.ds(j * L, L)] = (
                segc[pl.ds(j * L, L)] * 32 + ((j & 1) * L + lane_iota)
                + cbase)
        pltpu.async_copy(tok_hbm.at[idxc], buf, sem)
        pltpu.async_copy(chbm.at[cidxb], cbuf, sem)

    def wait_gathers(idxc, cidxb, buf, cbuf, sem):
        pltpu.make_async_copy(tok_hbm.at[idxc], buf, sem).wait()
        pltpu.make_async_copy(chbm.at[cidxb], cbuf, sem).wait()

    def compute_chunk(buf, cbuf, rowbase):
        def row_body(r, rcarry):
            rows = (r, r + 16)
            accs = [[jnp.zeros((L,), jnp.float32) for _ in range(NPART)]
                    for _ in rows]
            acc2s = [[jnp.zeros((L,), jnp.float32) for _ in range(NPART)]
                     for _ in rows]
            for k in range(NK):
                sl = pl.ds(k * L, L)
                p = k % NPART
                for g in range(2):
                    v = buf[rows[g], sl] + cbuf[rows[g], sl]
                    accs[g][p] = accs[g][p] + v
                    acc2s[g][p] = acc2s[g][p] + v * v
            means, invs = [], []
            for g in range(2):
                a = (accs[g][0] + accs[g][1]) + (accs[g][2] + accs[g][3])
                a2 = (acc2s[g][0] + acc2s[g][1]) + (acc2s[g][2] + acc2s[g][3])
                s1 = jnp.sum(a)
                s2 = jnp.sum(a2)
                mean = jnp.full((L,), s1, jnp.float32) * (1.0 / D)
                ex2 = jnp.full((L,), s2, jnp.float32) * (1.0 / D)
                means.append(mean)
                invs.append(_rsqrt_vec(ex2 - mean * mean + 1e-5))
            for k in range(NK):
                sl = pl.ds(k * L, L)
                g_ = gam[sl]
                b_ = bet[sl]
                for g in range(2):
                    v = buf[rows[g], sl] + cbuf[rows[g], sl]
                    u = (v - means[g]) * invs[g]
                    tbuf[rows[g], sl] = u * g_ + b_
            return rcarry

        lax.fori_loop(0, K_ROWS // 2, row_body, 0)
        pltpu.sync_copy(tbuf, out_hbm.at[pl.ds(rowbase, K_ROWS)])

    pltpu.sync_copy(x_hbm.at[pl.ds(base, K_ROWS)], idxcA)
    pltpu.sync_copy(seg_hbm.at[pl.ds(base, K_ROWS)], segcA)
    fire_gathers(idxcA, segcA, cidxbA, bufA, cbufA, semA)
    stage_idx(base + K_ROWS, idxcB, segcB)

    def step(m, carry):
        ra = base + (2 * m) * K_ROWS
        rb = ra + K_ROWS
        ra2 = lax.min(rb + K_ROWS, last_rb)
        rb2 = lax.min(ra2 + K_ROWS, last_rb)
        wait_idx(idxcB, segcB)
        fire_gathers(idxcB, segcB, cidxbB, bufB, cbufB, semB)
        wait_gathers(idxcA, cidxbA, bufA, cbufA, semA)
        stage_idx(ra2, idxcA, segcA)
        compute_chunk(bufA, cbufA, ra)
        wait_idx(idxcA, segcA)
        fire_gathers(idxcA, segcA, cidxbA, bufA, cbufA, semA)
        wait_gathers(idxcB, cidxbB, bufB, cbufB, semB)
        stage_idx(rb2, idxcB, segcB)
        compute_chunk(bufB, cbufB, rb)
        return carry

    lax.fori_loop(0, n_chunks // 2, step, 0)
    wait_idx(idxcB, segcB)
    wait_gathers(idxcA, cidxbA, bufA, cbufA, semA)


@jax.jit
def kernel(x, seg, tok_table, pos_table, seg_table, gamma, beta):
    info = plsc.get_sparse_core_info()
    n_workers = info.num_cores * info.num_subcores
    mesh = plsc.VectorSubcoreMesh(core_axis_name="c", subcore_axis_name="s")
    run = pl.kernel(
        functools.partial(_sc_body, n_workers),
        mesh=mesh,
        compiler_params=pltpu.CompilerParams(needs_layout_passes=False),
        out_type=(
            jax.ShapeDtypeStruct((N_TOK, D), jnp.float32),
            jax.ShapeDtypeStruct((n_workers * NCOMBO, D), jnp.float32),
        ),
        scratch_types=[
            pltpu.VMEM((K_ROWS,), jnp.int32),
            pltpu.VMEM((K_ROWS,), jnp.int32),
            pltpu.VMEM((K_ROWS,), jnp.int32),
            pltpu.VMEM((K_ROWS, D), jnp.float32),
            pltpu.VMEM((K_ROWS, D), jnp.float32),
            pltpu.VMEM((K_ROWS,), jnp.int32),
            pltpu.VMEM((K_ROWS,), jnp.int32),
            pltpu.VMEM((K_ROWS,), jnp.int32),
            pltpu.VMEM((K_ROWS, D), jnp.float32),
            pltpu.VMEM((K_ROWS, D), jnp.float32),
            pltpu.VMEM((K_ROWS, D), jnp.float32),
            pltpu.VMEM((D,), jnp.float32),
            pltpu.VMEM((D,), jnp.float32),
            pltpu.VMEM((D,), jnp.float32),
            pltpu.VMEM((2 * D,), jnp.float32),
            pltpu.SemaphoreType.DMA,
            pltpu.SemaphoreType.DMA,
            pltpu.SemaphoreType.DMA,
        ],
    )
    out, _ = run(x.reshape(-1), seg.reshape(-1),
                 tok_table, pos_table.reshape(-1), seg_table.reshape(-1),
                 gamma, beta)
    return out.reshape(x.shape[0], x.shape[1], D)


# speedup vs baseline: 3.7537x; 1.0076x over previous
import functools

import jax
import jax.numpy as jnp
from jax import lax
from jax.experimental import pallas as pl
from jax.experimental.pallas import tpu as pltpu
from jax.experimental.pallas import tpu_sc as plsc

D = 768
L = 16
NK = D // L
N_TOK = 4096 * 32
K_ROWS = 32
NPART = 4
NCOMBO = 64


def _rsqrt_vec(x):
    i = plsc.bitcast(x, jnp.int32)
    i = jnp.int32(0x5F3759DF) - lax.shift_right_logical(i, 1)
    y = plsc.bitcast(i, jnp.float32)
    for _ in range(3):
        y = y * (1.5 - 0.5 * x * y * y)
    return y


def _sc_body(n_workers, x_hbm, seg_hbm, tok_hbm, pos_hbm, segtab_hbm,
             gam_hbm, bet_hbm, out_hbm, chbm,
             idxcA, segcA, cidxbA, bufA, cbufA,
             idxcB, segcB, cidxbB, bufB, cbufB,
             tbuf, gam, bet, rowtmp, segrows, semA, semB, semI, semO):
    tpw = N_TOK // n_workers
    n_chunks = tpw // K_ROWS
    wid = lax.axis_index("s") * 2 + lax.axis_index("c")
    base = wid * tpw
    cbase = wid * (NCOMBO + K_ROWS)
    dummy_out = chbm.at[pl.ds(cbase + NCOMBO, K_ROWS)]
    last_rb = base + (n_chunks - 1) * K_ROWS
    lane_iota = lax.iota(jnp.int32, L)

    pltpu.sync_copy(gam_hbm, gam)
    pltpu.sync_copy(bet_hbm, bet)
    pltpu.sync_copy(segtab_hbm, segrows)

    def buildc(t, carry):
        pltpu.sync_copy(pos_hbm.at[pl.ds(t * D, D)], rowtmp)
        for s, dstref in ((0, bufA), (1, cbufA)):
            for k in range(NK):
                sl = pl.ds(k * L, L)
                dstref[t, sl] = rowtmp[sl] + segrows[pl.ds(s * D + k * L, L)]
        return carry

    lax.fori_loop(0, 32, buildc, 0)
    pltpu.sync_copy(bufA, chbm.at[pl.ds(cbase, K_ROWS)])
    pltpu.sync_copy(cbufA, chbm.at[pl.ds(cbase + K_ROWS, K_ROWS)])

    def stage_idx(rowbase, idxc, segc):
        c1 = pltpu.async_copy(x_hbm.at[pl.ds(rowbase, K_ROWS)], idxc, semI)
        c2 = pltpu.async_copy(seg_hbm.at[pl.ds(rowbase, K_ROWS)], segc, semI)
        return c1, c2

    def wait_idx(idxc, segc):
        pltpu.make_async_copy(x_hbm.at[pl.ds(0, K_ROWS)], idxc, semI).wait()
        pltpu.make_async_copy(seg_hbm.at[pl.ds(0, K_ROWS)], segc, semI).wait()

    def fire_gathers(idxc, segc, cidxb, buf, cbuf, sem):
        for j in range(K_ROWS // L):
            cidxb[pl.ds(j * L, L)] = (
                segc[pl.ds(j * L, L)] * 32 + ((j & 1) * L + lane_iota)
                + cbase)
        pltpu.async_copy(tok_hbm.at[idxc], buf, sem)
        pltpu.async_copy(chbm.at[cidxb], cbuf, sem)

    def wait_gathers(idxc, cidxb, buf, cbuf, sem):
        pltpu.make_async_copy(tok_hbm.at[idxc], buf, sem).wait()
        pltpu.make_async_copy(chbm.at[cidxb], cbuf, sem).wait()

    def compute_chunk(buf, cbuf, rowbase):
        pltpu.make_async_copy(tbuf, dummy_out, semO).wait()

        def row_body(r, rcarry):
            rows = (r, r + 16)
            accs = [[jnp.zeros((L,), jnp.float32) for _ in range(NPART)]
                    for _ in rows]
            acc2s = [[jnp.zeros((L,), jnp.float32) for _ in range(NPART)]
                     for _ in rows]
            for k in range(NK):
                sl = pl.ds(k * L, L)
                p = k % NPART
                for g in range(2):
                    v = buf[rows[g], sl] + cbuf[rows[g], sl]
                    accs[g][p] = accs[g][p] + v
                    acc2s[g][p] = acc2s[g][p] + v * v
            means, invs = [], []
            for g in range(2):
                a = (accs[g][0] + accs[g][1]) + (accs[g][2] + accs[g][3])
                a2 = (acc2s[g][0] + acc2s[g][1]) + (acc2s[g][2] + acc2s[g][3])
                s1 = jnp.sum(a)
                s2 = jnp.sum(a2)
                mean = jnp.full((L,), s1, jnp.float32) * (1.0 / D)
                ex2 = jnp.full((L,), s2, jnp.float32) * (1.0 / D)
                means.append(mean)
                invs.append(_rsqrt_vec(ex2 - mean * mean + 1e-5))
            for k in range(NK):
                sl = pl.ds(k * L, L)
                g_ = gam[sl]
                b_ = bet[sl]
                for g in range(2):
                    v = buf[rows[g], sl] + cbuf[rows[g], sl]
                    u = (v - means[g]) * invs[g]
                    tbuf[rows[g], sl] = u * g_ + b_
            return rcarry

        lax.fori_loop(0, K_ROWS // 2, row_body, 0)
        pltpu.async_copy(tbuf, out_hbm.at[pl.ds(rowbase, K_ROWS)], semO)

    pltpu.async_copy(tbuf, dummy_out, semO)
    pltpu.sync_copy(x_hbm.at[pl.ds(base, K_ROWS)], idxcA)
    pltpu.sync_copy(seg_hbm.at[pl.ds(base, K_ROWS)], segcA)
    fire_gathers(idxcA, segcA, cidxbA, bufA, cbufA, semA)
    stage_idx(base + K_ROWS, idxcB, segcB)

    def step(m, carry):
        ra = base + (2 * m) * K_ROWS
        rb = ra + K_ROWS
        ra2 = lax.min(rb + K_ROWS, last_rb)
        rb2 = lax.min(ra2 + K_ROWS, last_rb)
        wait_idx(idxcB, segcB)
        fire_gathers(idxcB, segcB, cidxbB, bufB, cbufB, semB)
        wait_gathers(idxcA, cidxbA, bufA, cbufA, semA)
        stage_idx(ra2, idxcA, segcA)
        compute_chunk(bufA, cbufA, ra)
        wait_idx(idxcA, segcA)
        fire_gathers(idxcA, segcA, cidxbA, bufA, cbufA, semA)
        wait_gathers(idxcB, cidxbB, bufB, cbufB, semB)
        stage_idx(rb2, idxcB, segcB)
        compute_chunk(bufB, cbufB, rb)
        return carry

    lax.fori_loop(0, n_chunks // 2, step, 0)
    wait_idx(idxcB, segcB)
    wait_gathers(idxcA, cidxbA, bufA, cbufA, semA)
    pltpu.make_async_copy(tbuf, dummy_out, semO).wait()


@jax.jit
def kernel(x, seg, tok_table, pos_table, seg_table, gamma, beta):
    info = plsc.get_sparse_core_info()
    n_workers = info.num_cores * info.num_subcores
    mesh = plsc.VectorSubcoreMesh(core_axis_name="c", subcore_axis_name="s")
    run = pl.kernel(
        functools.partial(_sc_body, n_workers),
        mesh=mesh,
        compiler_params=pltpu.CompilerParams(needs_layout_passes=False),
        out_type=(
            jax.ShapeDtypeStruct((N_TOK, D), jnp.float32),
            jax.ShapeDtypeStruct((n_workers * (NCOMBO + K_ROWS), D),
                                 jnp.float32),
        ),
        scratch_types=[
            pltpu.VMEM((K_ROWS,), jnp.int32),
            pltpu.VMEM((K_ROWS,), jnp.int32),
            pltpu.VMEM((K_ROWS,), jnp.int32),
            pltpu.VMEM((K_ROWS, D), jnp.float32),
            pltpu.VMEM((K_ROWS, D), jnp.float32),
            pltpu.VMEM((K_ROWS,), jnp.int32),
            pltpu.VMEM((K_ROWS,), jnp.int32),
            pltpu.VMEM((K_ROWS,), jnp.int32),
            pltpu.VMEM((K_ROWS, D), jnp.float32),
            pltpu.VMEM((K_ROWS, D), jnp.float32),
            pltpu.VMEM((K_ROWS, D), jnp.float32),
            pltpu.VMEM((D,), jnp.float32),
            pltpu.VMEM((D,), jnp.float32),
            pltpu.VMEM((D,), jnp.float32),
            pltpu.VMEM((2 * D,), jnp.float32),
            pltpu.SemaphoreType.DMA,
            pltpu.SemaphoreType.DMA,
            pltpu.SemaphoreType.DMA,
            pltpu.SemaphoreType.DMA,
        ],
    )
    out, _ = run(x.reshape(-1), seg.reshape(-1),
                 tok_table, pos_table.reshape(-1), seg_table.reshape(-1),
                 gamma, beta)
    return out.reshape(x.shape[0], x.shape[1], D)
